# baseline (speedup 1.0000x reference)
import functools

import jax
import jax.numpy as jnp
from jax import lax
from jax.experimental import pallas as pl
from jax.experimental.pallas import tpu as pltpu
from jax.experimental.pallas import tpu_sc as plsc

_NUM_BINS = 1000
_NUM_FEATURES = 64
_N = 819200

_LANES = 16
_CROWS = 256


def _minmax_body(x_ref, o_ref):
    x = x_ref[...]
    row = lax.broadcasted_iota(jnp.int32, (8, 128), 0)
    col = lax.broadcasted_iota(jnp.int32, (8, 128), 1)
    o_ref[...] = jnp.where(
        (row == 0) & (col == 0),
        jnp.min(x),
        jnp.where((row == 0) & (col == 1), jnp.max(x), 0.0),
    )


def _minmax(values):
    v2 = values.reshape(6400, 128)
    return pl.pallas_call(
        _minmax_body,
        out_shape=jax.ShapeDtypeStruct((8, 128), jnp.float32),
    )(v2)


def _make_sc_kernel():
    info = plsc.get_sparse_core_info()
    nc, ns = info.num_cores, info.num_subcores
    nw = nc * ns
    rows = _N // nw
    nchunks = rows // _CROWS

    mesh = plsc.VectorSubcoreMesh(core_axis_name="c", subcore_axis_name="s")

    @functools.partial(
        pl.kernel,
        mesh=mesh,
        compiler_params=pltpu.CompilerParams(
            needs_layout_passes=False, use_tc_tiling_on_sc=False
        ),
        out_type=jax.ShapeDtypeStruct((_N, _NUM_FEATURES), jnp.float32),
        scratch_types=[
            pltpu.VMEM((_NUM_BINS + 8,), jnp.float32),
            pltpu.VMEM((2, 128), jnp.float32),
            pltpu.VMEM((rows,), jnp.float32),
            pltpu.VMEM((_NUM_BINS, _NUM_FEATURES), jnp.float32),
            pltpu.VMEM((_CROWS,), jnp.int32),
            pltpu.VMEM((2, _CROWS, _NUM_FEATURES), jnp.float32),
            pltpu.SemaphoreType.DMA,
        ],
    )
    def sc_kernel(values_hbm, table_hbm, edges_hbm, params_hbm, out_hbm,
                  edges_v, params_v, vals_v, table_v, idx_v, rows_v, ssem):
        wid = lax.axis_index("s") * nc + lax.axis_index("c")
        base = wid * rows

        pltpu.sync_copy(edges_hbm, edges_v)
        pltpu.sync_copy(params_hbm, params_v)
        pltpu.sync_copy(table_hbm, table_v)
        pltpu.sync_copy(values_hbm.at[pl.ds(base, rows)], vals_v)

        mnv = params_v[0, pl.ds(0, _LANES)]
        inv = params_v[1, pl.ds(0, _LANES)]

        def bin_body(i, carry):
            off = i * _LANES
            v = vals_v[pl.ds(off, _LANES)]
            t = (v - mnv) * inv
            b = jnp.clip(t.astype(jnp.int32), 0, _NUM_BINS - 1)
            for _ in range(2):
                e_lo = plsc.load_gather(edges_v, [b])
                e_hi = plsc.load_gather(edges_v, [b + 1])
                b = b + (v >= e_hi).astype(jnp.int32) - (v < e_lo).astype(jnp.int32)
                b = jnp.clip(b, 0, _NUM_BINS - 1)
            idx_v[pl.ds((i % (_CROWS // _LANES)) * _LANES, _LANES)] = b
            return carry

        def bin_chunk(c):
            base_i = c * (_CROWS // _LANES)
            lax.fori_loop(
                0,
                _CROWS // _LANES,
                lambda i, cc: bin_body(base_i + i, cc),
                0,
                unroll=8,
            )

        def assemble_chunk(bank):
            bank_ref = rows_v.at[bank]
            iota = lax.iota(jnp.int32, _LANES)

            def group_body(gi, carry):
                bvec = idx_v[pl.ds(gi * _LANES, _LANES)]
                rvec = iota + gi * _LANES
                for c in range(_NUM_FEATURES):
                    cc = iota * 0 + c
                    w = plsc.load_gather(table_v, [bvec, cc])
                    plsc.store_scatter(bank_ref, [rvec, cc], w)
                return carry

            lax.fori_loop(0, _CROWS // _LANES, group_body, 0, unroll=2)

        def store_cp(c, bank):
            return pltpu.make_async_copy(
                rows_v.at[bank],
                out_hbm.at[pl.ds(base + c * _CROWS, _CROWS)],
                ssem,
            )

        def pipe_body(k, carry):
            c0 = 2 * k
            @pl.when(k > 0)
            def _():
                store_cp(c0 - 2, 0).wait()

            bin_chunk(c0)
            assemble_chunk(0)
            store_cp(c0, 0).start()

            @pl.when(k > 0)
            def _():
                store_cp(c0 - 1, 1).wait()

            bin_chunk(c0 + 1)
            assemble_chunk(1)
            store_cp(c0 + 1, 1).start()
            return carry

        lax.fori_loop(0, nchunks // 2, pipe_body, 0, unroll=False)

        store_cp(nchunks - 2, 0).wait()
        store_cp(nchunks - 1, 1).wait()

    return sc_kernel


def kernel(values, token_emb):
    mm = _minmax(values)
    mn = mm[0, 0]
    mx = mm[0, 1]
    rng = mx - mn
    adj = rng * 0.001
    edges = jnp.linspace(mn, mx, _NUM_BINS + 1)
    edges = edges.at[0].add(-adj)
    edges = edges.at[-1].add(adj)
    edges_p = jnp.concatenate([edges, jnp.full((7,), edges[-1], jnp.float32)])
    inv = jnp.float32(_NUM_BINS) / rng
    params = jnp.stack(
        [jnp.full((128,), mn, jnp.float32), jnp.full((128,), inv, jnp.float32)]
    )
    sc = _make_sc_kernel()
    return sc(values, token_emb, edges_p, params)


# speedup vs baseline: 2.8996x; 2.8996x over previous
import functools

import jax
import jax.numpy as jnp
from jax import lax
from jax.experimental import pallas as pl
from jax.experimental.pallas import tpu as pltpu
from jax.experimental.pallas import tpu_sc as plsc

_NUM_BINS = 1000
_NUM_FEATURES = 64
_N = 819200

_LANES = 16
_CROWS = 128


def _minmax_body(x_ref, o_ref):
    x = x_ref[...]
    row = lax.broadcasted_iota(jnp.int32, (8, 128), 0)
    col = lax.broadcasted_iota(jnp.int32, (8, 128), 1)
    o_ref[...] = jnp.where(
        (row == 0) & (col == 0),
        jnp.min(x),
        jnp.where((row == 0) & (col == 1), jnp.max(x), 0.0),
    )


def _minmax(values):
    v2 = values.reshape(6400, 128)
    return pl.pallas_call(
        _minmax_body,
        out_shape=jax.ShapeDtypeStruct((8, 128), jnp.float32),
    )(v2)


def _make_sc_kernel():
    info = plsc.get_sparse_core_info()
    nc, ns = info.num_cores, info.num_subcores
    nw = nc * ns
    rows = _N // nw
    half = rows // 2
    nchunks = half // _CROWS

    mesh = plsc.VectorSubcoreMesh(core_axis_name="c", subcore_axis_name="s")

    @functools.partial(
        pl.kernel,
        mesh=mesh,
        compiler_params=pltpu.CompilerParams(
            needs_layout_passes=False, use_tc_tiling_on_sc=False
        ),
        out_type=jax.ShapeDtypeStruct((_N, _NUM_FEATURES), jnp.float32),
        scratch_types=[
            pltpu.VMEM((_NUM_BINS + 8,), jnp.float32),
            pltpu.VMEM((2, 128), jnp.float32),
            pltpu.VMEM((rows,), jnp.float32),
            pltpu.VMEM((_NUM_BINS, _NUM_FEATURES), jnp.float32),
            pltpu.VMEM((2, _CROWS), jnp.int32),
            pltpu.VMEM((_CROWS,), jnp.int32),
            pltpu.VMEM((2, _CROWS, _NUM_FEATURES), jnp.float32),
            pltpu.VMEM((2, _CROWS, _NUM_FEATURES), jnp.float32),
            pltpu.SemaphoreType.DMA,
            pltpu.SemaphoreType.DMA,
        ],
    )
    def sc_kernel(values_hbm, table_hbm, edges_hbm, params_hbm, out_hbm,
                  edges_v, params_v, vals_v, table_v, idxg_v, idxa_v,
                  growa_v, arows_v, gsem, ssem):
        wid = lax.axis_index("s") * nc + lax.axis_index("c")
        base = wid * rows

        pltpu.sync_copy(edges_hbm, edges_v)
        pltpu.sync_copy(params_hbm, params_v)
        pltpu.sync_copy(table_hbm, table_v)
        pltpu.sync_copy(values_hbm.at[pl.ds(base, rows)], vals_v)

        mnv = params_v[0, pl.ds(0, _LANES)]
        inv = params_v[1, pl.ds(0, _LANES)]

        def bin_block(row0, dst_ref):
            def bin_body(i, carry):
                v = vals_v[pl.ds(row0 + i * _LANES, _LANES)]
                t = (v - mnv) * inv
                b = jnp.clip(t.astype(jnp.int32), 0, _NUM_BINS - 1)
                for _ in range(2):
                    e_lo = plsc.load_gather(edges_v, [b])
                    e_hi = plsc.load_gather(edges_v, [b + 1])
                    b = (
                        b
                        + (v >= e_hi).astype(jnp.int32)
                        - (v < e_lo).astype(jnp.int32)
                    )
                    b = jnp.clip(b, 0, _NUM_BINS - 1)
                dst_ref[pl.ds(i * _LANES, _LANES)] = b
                return carry

            lax.fori_loop(0, _CROWS // _LANES, bin_body, 0, unroll=8)

        def assemble_chunk(bank):
            def group_body(gi, carry):
                bvec = idxa_v[pl.ds(gi * _LANES, _LANES)]
                for j in range(_LANES):
                    b = bvec[j]
                    r = gi * _LANES + j
                    for k in range(_NUM_FEATURES // _LANES):
                        arows_v[bank, r, pl.ds(k * _LANES, _LANES)] = (
                            table_v[b, pl.ds(k * _LANES, _LANES)]
                        )
                return carry

            lax.fori_loop(0, _CROWS // _LANES, group_body, 0, unroll=2)

        def gather_cp(c, bank):
            return pltpu.make_async_copy(
                table_hbm.at[idxg_v.at[bank]],
                growa_v.at[bank],
                gsem,
            )

        def gstore_cp(c, bank):
            return pltpu.make_async_copy(
                growa_v.at[bank],
                out_hbm.at[pl.ds(base + c * _CROWS, _CROWS)],
                ssem,
            )

        def astore_cp(c, bank):
            return pltpu.make_async_copy(
                arows_v.at[bank],
                out_hbm.at[pl.ds(base + half + c * _CROWS, _CROWS)],
                ssem,
            )

        def sub_iter(p, bank):
            @pl.when(p >= 2)
            def _():
                gstore_cp(p - 2, bank).wait()

            bin_block(p * _CROWS, idxg_v.at[bank])
            gather_cp(p, bank).start()

            @pl.when(p >= 2)
            def _():
                astore_cp(p - 2, bank).wait()

            bin_block(half + p * _CROWS, idxa_v)
            assemble_chunk(bank)
            astore_cp(p, bank).start()

            gather_cp(p, bank).wait()
            gstore_cp(p, bank).start()

        def pipe_body(k, carry):
            p0 = 2 * k
            sub_iter(p0, 0)
            sub_iter(p0 + 1, 1)
            return carry

        lax.fori_loop(0, nchunks // 2, pipe_body, 0, unroll=False)

        gstore_cp(nchunks - 2, 0).wait()
        gstore_cp(nchunks - 1, 1).wait()
        astore_cp(nchunks - 2, 0).wait()
        astore_cp(nchunks - 1, 1).wait()

    return sc_kernel


def kernel(values, token_emb):
    mm = _minmax(values)
    mn = mm[0, 0]
    mx = mm[0, 1]
    rng = mx - mn
    adj = rng * 0.001
    edges = jnp.linspace(mn, mx, _NUM_BINS + 1)
    edges = edges.at[0].add(-adj)
    edges = edges.at[-1].add(adj)
    edges_p = jnp.concatenate([edges, jnp.full((7,), edges[-1], jnp.float32)])
    inv = jnp.float32(_NUM_BINS) / rng
    params = jnp.stack(
        [jnp.full((128,), mn, jnp.float32), jnp.full((128,), inv, jnp.float32)]
    )
    sc = _make_sc_kernel()
    return sc(values, token_emb, edges_p, params)


# speedup vs baseline: 2.9049x; 1.0018x over previous
import functools

import jax
import jax.numpy as jnp
from jax import lax
from jax.experimental import pallas as pl
from jax.experimental.pallas import tpu as pltpu
from jax.experimental.pallas import tpu_sc as plsc

_NUM_BINS = 1000
_NUM_FEATURES = 64
_N = 819200

_LANES = 16
_CROWS = 128


def _minmax_body(x_ref, o_ref):
    x = x_ref[...]
    row = lax.broadcasted_iota(jnp.int32, (8, 128), 0)
    col = lax.broadcasted_iota(jnp.int32, (8, 128), 1)
    o_ref[...] = jnp.where(
        (row == 0) & (col == 0),
        jnp.min(x),
        jnp.where((row == 0) & (col == 1), jnp.max(x), 0.0),
    )


def _minmax(values):
    v2 = values.reshape(6400, 128)
    return pl.pallas_call(
        _minmax_body,
        out_shape=jax.ShapeDtypeStruct((8, 128), jnp.float32),
    )(v2)


def _make_sc_kernel():
    info = plsc.get_sparse_core_info()
    nc, ns = info.num_cores, info.num_subcores
    nw = nc * ns
    rows = _N // nw
    half = rows // 2
    nchunks = half // _CROWS

    mesh = plsc.VectorSubcoreMesh(core_axis_name="c", subcore_axis_name="s")

    @functools.partial(
        pl.kernel,
        mesh=mesh,
        compiler_params=pltpu.CompilerParams(
            needs_layout_passes=False, use_tc_tiling_on_sc=False
        ),
        out_type=jax.ShapeDtypeStruct((_N, _NUM_FEATURES), jnp.float32),
        scratch_types=[
            pltpu.VMEM((_NUM_BINS + 8,), jnp.float32),
            pltpu.VMEM((2, 128), jnp.float32),
            pltpu.VMEM((rows,), jnp.float32),
            pltpu.VMEM((_NUM_BINS, _NUM_FEATURES), jnp.float32),
            pltpu.VMEM((2, _CROWS), jnp.int32),
            pltpu.VMEM((_CROWS,), jnp.int32),
            pltpu.VMEM((2, _CROWS, _NUM_FEATURES), jnp.float32),
            pltpu.VMEM((2, _CROWS, _NUM_FEATURES), jnp.float32),
            pltpu.SemaphoreType.DMA,
            pltpu.SemaphoreType.DMA,
        ],
    )
    def sc_kernel(values_hbm, table_hbm, edges_hbm, params_hbm, out_hbm,
                  edges_v, params_v, vals_v, table_v, idxg_v, idxa_v,
                  growa_v, arows_v, gsem, ssem):
        wid = lax.axis_index("s") * nc + lax.axis_index("c")
        base = wid * rows

        pltpu.sync_copy(edges_hbm, edges_v)
        pltpu.sync_copy(params_hbm, params_v)
        pltpu.sync_copy(table_hbm, table_v)
        pltpu.sync_copy(values_hbm.at[pl.ds(base, rows)], vals_v)

        mnv = params_v[0, pl.ds(0, _LANES)]
        inv = params_v[1, pl.ds(0, _LANES)]

        def bin_block(row0, dst_ref):
            def bin_body(i, carry):
                v = vals_v[pl.ds(row0 + i * _LANES, _LANES)]
                t = (v - mnv) * inv
                b = jnp.clip(t.astype(jnp.int32), 0, _NUM_BINS - 1)
                for _ in range(2):
                    e_lo = plsc.load_gather(edges_v, [b])
                    e_hi = plsc.load_gather(edges_v, [b + 1])
                    b = (
                        b
                        + (v >= e_hi).astype(jnp.int32)
                        - (v < e_lo).astype(jnp.int32)
                    )
                    b = jnp.clip(b, 0, _NUM_BINS - 1)
                dst_ref[pl.ds(i * _LANES, _LANES)] = b
                return carry

            lax.fori_loop(0, _CROWS // _LANES, bin_body, 0, unroll=8)

        def assemble_chunk(bank):
            def group_body(gi, carry):
                bvec = idxa_v[pl.ds(gi * _LANES, _LANES)]
                for j in range(_LANES):
                    b = bvec[j]
                    r = gi * _LANES + j
                    for k in range(_NUM_FEATURES // _LANES):
                        arows_v[bank, r, pl.ds(k * _LANES, _LANES)] = (
                            table_v[b, pl.ds(k * _LANES, _LANES)]
                        )
                return carry

            lax.fori_loop(0, _CROWS // _LANES, group_body, 0, unroll=2)

        def gather_cp(c, bank):
            return pltpu.make_async_copy(
                table_hbm.at[idxg_v.at[bank]],
                growa_v.at[bank],
                gsem,
            )

        def gstore_cp(c, bank):
            return pltpu.make_async_copy(
                growa_v.at[bank],
                out_hbm.at[pl.ds(base + c * _CROWS, _CROWS)],
                ssem,
            )

        def astore_cp(c, bank):
            return pltpu.make_async_copy(
                arows_v.at[bank],
                out_hbm.at[pl.ds(base + half + c * _CROWS, _CROWS)],
                ssem,
            )

        def sub_iter(p, bank):
            @pl.when(p >= 1)
            def _():
                other = 1 - bank
                gather_cp(p - 1, other).wait()
                gstore_cp(p - 1, other).start()

            @pl.when(p >= 2)
            def _():
                gstore_cp(p - 2, bank).wait()

            bin_block(p * _CROWS, idxg_v.at[bank])
            gather_cp(p, bank).start()

            @pl.when(p >= 2)
            def _():
                astore_cp(p - 2, bank).wait()

            bin_block(half + p * _CROWS, idxa_v)
            assemble_chunk(bank)
            astore_cp(p, bank).start()

        def pipe_body(k, carry):
            p0 = 2 * k
            sub_iter(p0, 0)
            sub_iter(p0 + 1, 1)
            return carry

        lax.fori_loop(0, nchunks // 2, pipe_body, 0, unroll=False)

        gather_cp(nchunks - 1, 1).wait()
        gstore_cp(nchunks - 1, 1).start()
        gstore_cp(nchunks - 2, 0).wait()
        gstore_cp(nchunks - 1, 1).wait()
        astore_cp(nchunks - 2, 0).wait()
        astore_cp(nchunks - 1, 1).wait()

    return sc_kernel


def kernel(values, token_emb):
    mm = _minmax(values)
    mn = mm[0, 0]
    mx = mm[0, 1]
    rng = mx - mn
    adj = rng * 0.001
    edges = jnp.linspace(mn, mx, _NUM_BINS + 1)
    edges = edges.at[0].add(-adj)
    edges = edges.at[-1].add(adj)
    edges_p = jnp.concatenate([edges, jnp.full((7,), edges[-1], jnp.float32)])
    inv = jnp.float32(_NUM_BINS) / rng
    params = jnp.stack(
        [jnp.full((128,), mn, jnp.float32), jnp.full((128,), inv, jnp.float32)]
    )
    sc = _make_sc_kernel()
    return sc(values, token_emb, edges_p, params)
